# baseline (speedup 1.0000x reference)
import functools
import math

import jax
import jax.numpy as jnp
from jax import lax
from jax.experimental import pallas as pl
from jax.experimental.pallas import tpu as pltpu
from jax.experimental.pallas import tpu_sc as plsc

EPS = 1e-9
N = 10000
D = 128
E = 320000
SCALE = 1.0 / math.sqrt(128.0)

NC = 2
NS = 16
NW = NC * NS
EPT = E // NW
CH = 40
NCHUNK = EPT // CH
NCOMP = 252
PADC = 260
NPAD = 10240
RPT = NPAD // NS


_GDN = lax.GatherDimensionNumbers(offset_dims=(), collapsed_slice_dims=(0,),
                                  start_index_map=(0,))


def _shuffle(p, idx):
    return lax.gather(p, idx[:, None], _GDN, (1,),
                      mode=lax.GatherScatterMode.PROMISE_IN_BOUNDS)


def _lanesum(p, lane):
    for sh in (8, 4, 2, 1):
        p = p + _shuffle(p, lane ^ sh)
    return p


def _mink_sign(shape):
    col = lax.broadcasted_iota(jnp.int32, shape, 1)
    return jnp.where(col == D - 1, -1.0, 1.0).astype(jnp.float32)


def _row_normalize(a):
    at = a[:, D - 1:D]
    ss = jnp.maximum(jnp.sum(a * a, axis=1, keepdims=True) - at * at, 0.0)
    inv = 1.0 / jnp.maximum(jnp.sqrt(ss), EPS)
    col = lax.broadcasted_iota(jnp.int32, a.shape, 1)
    return jnp.where(col == D - 1, a, a * inv)


def _prep_body(x_ref, wq_ref, bq_ref, wk_ref, bk_ref, wv_ref, bv_ref,
               qn_ref, knm_ref, val_ref, cr_ref):
    x = x_ref[...]
    sgn = _mink_sign((1, D))
    a, b, c, d = x[0:1], x[1:2], x[2:3], x[3:4]
    ac = jnp.sum(a * c * sgn)
    bd = jnp.sum(b * d * sgn)
    ad = jnp.sum(a * d * sgn)
    bc = jnp.sum(b * c * sgn)
    cr_ref[...] = jnp.reshape((ac * bd) / (ad * bc + EPS), (1, 1))

    xp = _row_normalize(x)
    q = jnp.dot(xp, wq_ref[...], preferred_element_type=jnp.float32) + bq_ref[...]
    k = jnp.dot(xp, wk_ref[...], preferred_element_type=jnp.float32) + bk_ref[...]
    v = jnp.dot(xp, wv_ref[...], preferred_element_type=jnp.float32) + bv_ref[...]
    qn_ref[...] = _row_normalize(q)
    kn = _row_normalize(k)
    col = lax.broadcasted_iota(jnp.int32, kn.shape, 1)
    knm_ref[...] = jnp.where(col == D - 1, -kn, kn) * SCALE
    val_ref[...] = v


@functools.partial(jax.jit, static_argnums=())
def _prep(x, Wq, bq, Wk, bk, Wv, bv):
    return pl.pallas_call(
        _prep_body,
        out_shape=[
            jax.ShapeDtypeStruct((N, D), jnp.float32),
            jax.ShapeDtypeStruct((N, D), jnp.float32),
            jax.ShapeDtypeStruct((N, D), jnp.float32),
            jax.ShapeDtypeStruct((1, 1), jnp.float32),
        ],
    )(x, Wq, bq, Wk, bk, Wv, bv)


def _edge_kernel(qn_hbm, knm_hbm, val_hbm, eidx_hbm,
                 acc_hbm, sums_hbm,
                 acc_sp,
                 ic0, ic1, ic2, ic3, ic4, ic5,
                 qb0, kb0, vb0, qb1, kb1, vb1, qb2, kb2, vb2, sbuf,
                 si0, si1, si2, si3, si4, si5,
                 sq0, sk0, sv0, sq1, sk1, sv1, sq2, sk2, sv2,
                 ss0, ss1, ss2):
    cid = lax.axis_index("c")
    sid = lax.axis_index("s")
    wid = cid * NS + sid
    ics = (ic0, ic1, ic2, ic3, ic4, ic5)
    isems = (si0, si1, si2, si3, si4, si5)
    qbufs, kbufs, vbufs = (qb0, qb1, qb2), (kb0, kb1, kb2), (vb0, vb1, vb2)
    dsems = ((sq0, sk0, sv0), (sq1, sk1, sv1), (sq2, sk2, sv2))
    ssems = (ss0, ss1, ss2)

    zrow = jnp.zeros((16,), jnp.float32)

    def zb(i, carry):
        for j in range(D // 16):
            qb0[i, pl.ds(j * 16, 16)] = zrow
        return carry

    lax.fori_loop(0, CH, zb, 0)
    for t in range(RPT // CH):
        pltpu.sync_copy(qb0, acc_sp.at[pl.ds(sid * RPT + t * CH, CH)])
    plsc.subcore_barrier()

    lane = lax.iota(jnp.int32, 16)

    def issue_idx(g, b6):
        pltpu.async_copy(eidx_hbm.at[wid, g], ics[b6], isems[b6])

    def wait_idx(g, b6):
        pltpu.make_async_copy(eidx_hbm.at[wid, g], ics[b6], isems[b6]).wait()

    def issue_data(g, b3, b6):
        ic = ics[b6]
        pltpu.async_copy(qn_hbm.at[ic.at[0]], qbufs[b3], dsems[b3][0])
        pltpu.async_copy(knm_hbm.at[ic.at[1]], kbufs[b3], dsems[b3][1])
        pltpu.async_copy(val_hbm.at[ic.at[1]], vbufs[b3], dsems[b3][2])

    def wait_data(g, b3, b6):
        ic = ics[b6]
        pltpu.make_async_copy(qn_hbm.at[ic.at[0]], qbufs[b3], dsems[b3][0]).wait()
        pltpu.make_async_copy(knm_hbm.at[ic.at[1]], kbufs[b3], dsems[b3][1]).wait()
        pltpu.make_async_copy(val_hbm.at[ic.at[1]], vbufs[b3], dsems[b3][2]).wait()

    def wait_scatter(b3, b6):
        pltpu.make_async_copy(vbufs[b3], acc_sp.at[ics[b6].at[0]],
                              ssems[b3]).wait()

    def compute(g, b3, b6, valid, lsum):
        qbuf, kbuf, vbuf = qbufs[b3], kbufs[b3], vbufs[b3]

        def edot(e, ls):
            p = qbuf[e, pl.ds(0, 16)] * kbuf[e, pl.ds(0, 16)]
            for j in range(1, D // 16):
                p = p + qbuf[e, pl.ds(j * 16, 16)] * kbuf[e, pl.ds(j * 16, 16)]
            w = jnp.exp(_lanesum(p, lane))
            w = jnp.where(valid, w, 0.0)
            for j in range(D // 16):
                vbuf[e, pl.ds(j * 16, 16)] = vbuf[e, pl.ds(j * 16, 16)] * w
            return ls + w

        lsum = plsc.parallel_loop(0, CH, 1, unroll=4, carry=lsum)(edot)
        pltpu.async_copy(vbuf, acc_sp.at[ics[b6].at[0]], ssems[b3], add=True)
        return lsum

    for g0 in range(5):
        issue_idx(g0, g0)
    wait_idx(0, 0)
    issue_data(0, 0, 0)
    wait_idx(1, 1)
    issue_data(1, 1, 1)

    def hexa(go, lsum):
        g0 = go * 6
        for c in range(6):
            g = g0 + c
            b3, b6 = c % 3, c

            @pl.when(g >= 1)
            def _():
                wait_scatter((c + 2) % 3, (c + 5) % 6)

            wait_idx(g + 2, (c + 2) % 6)
            issue_data(g + 2, (c + 2) % 3, (c + 2) % 6)
            wait_data(g, b3, b6)
            lsum = compute(g, b3, b6, g < NCHUNK, lsum)
            issue_idx(g + 5, (c + 5) % 6)
        return lsum

    lsum = lax.fori_loop(0, NCOMP // 6, hexa, jnp.zeros((16,), jnp.float32))

    wait_idx(NCOMP + 2, (NCOMP + 2) % 6)
    wait_idx(NCOMP + 3, (NCOMP + 3) % 6)
    wait_idx(NCOMP + 4, (NCOMP + 4) % 6)
    wait_data(NCOMP, NCOMP % 3, NCOMP % 6)
    wait_data(NCOMP + 1, (NCOMP + 1) % 3, (NCOMP + 1) % 6)
    wait_scatter((NCOMP - 1) % 3, (NCOMP - 1) % 6)

    sbuf[:] = lsum
    pltpu.sync_copy(sbuf, sums_hbm.at[pl.ds(wid * 16, 16)])

    plsc.subcore_barrier()
    for t in range(RPT // CH):
        sl = pl.ds(sid * RPT + t * CH, CH)
        pltpu.sync_copy(acc_sp.at[sl], acc_hbm.at[cid, sl])


def _edge(qn, knm, vals, rows, cols):
    mesh = plsc.VectorSubcoreMesh(core_axis_name="c", subcore_axis_name="s")
    f = functools.partial(
        pl.kernel,
        mesh=mesh,
        out_type=[
            jax.ShapeDtypeStruct((NC, NPAD, D), jnp.float32),
            jax.ShapeDtypeStruct((NW * 16,), jnp.float32),
        ],
        scratch_types=(
            [pltpu.VMEM_SHARED((NPAD, D), jnp.float32)]
            + [pltpu.VMEM((2, CH), jnp.int32)] * 6
            + [pltpu.VMEM((CH, D), jnp.float32)] * 9
            + [pltpu.VMEM((16,), jnp.float32)]
            + [pltpu.SemaphoreType.DMA] * 18
        ),
    )(_edge_kernel)
    eidx = jnp.stack([rows.reshape(NW, NCHUNK, CH),
                      cols.reshape(NW, NCHUNK, CH)], axis=2)
    eidx = jnp.pad(eidx, ((0, 0), (0, PADC - NCHUNK), (0, 0), (0, 0)))
    return f(qn, knm, vals, eidx)


def _fin_body(acc_ref, sums_ref, wo_ref, bo_ref, cr_ref, out_ref):
    A = acc_ref[0, 0:N, :] + acc_ref[1, 0:N, :]
    Z = jnp.sum(sums_ref[...][:, 0:1])
    o = (jnp.dot(A, wo_ref[...], preferred_element_type=jnp.float32) * (1.0 / Z)
         + bo_ref[...])
    sgn = _mink_sign((1, D))
    a, b, c, d = o[0:1], o[1:2], o[2:3], o[3:4]
    ac = jnp.sum(a * c * sgn)
    bd = jnp.sum(b * d * sgn)
    ad = jnp.sum(a * d * sgn)
    bc = jnp.sum(b * c * sgn)
    cr_now = (ac * bd) / (ad * bc + EPS)
    tgt = cr_ref[0, 0]
    scale = jnp.where(jnp.abs(cr_now) > EPS,
                      jnp.sqrt(jnp.abs(tgt / (cr_now + EPS))),
                      1.0)
    out_ref[...] = o * scale


def _finish(acc, sums, Wo, bo, cr):
    return pl.pallas_call(
        _fin_body,
        out_shape=jax.ShapeDtypeStruct((N, D), jnp.float32),
    )(acc, sums, Wo, bo, cr)


def kernel(x, edge_index, Wq, bq, Wk, bk, Wv, bv, Wo, bo):
    rows = edge_index[0].astype(jnp.int32)
    cols = edge_index[1].astype(jnp.int32)
    qn, knm, vals, cr = _prep(x, Wq, bq.reshape(1, D), Wk, bk.reshape(1, D),
                              Wv, bv.reshape(1, D))
    acc, sums = _edge(qn, knm, vals, rows, cols)
    return _finish(acc, sums.reshape(NW, 16), Wo, bo.reshape(1, D), cr)


# speedup vs baseline: 1.0401x; 1.0401x over previous
import functools
import math

import jax
import jax.numpy as jnp
from jax import lax
from jax.experimental import pallas as pl
from jax.experimental.pallas import tpu as pltpu
from jax.experimental.pallas import tpu_sc as plsc

EPS = 1e-9
N = 10000
D = 128
E = 320000
SCALE = 1.0 / math.sqrt(128.0)

NC = 2
NS = 16
NW = NC * NS
EPT = E // NW
CH = 40
NCHUNK = EPT // CH
NCOMP = 252
PADC = 260
NPAD = 10240
RPT = NPAD // NS


_GDN = lax.GatherDimensionNumbers(offset_dims=(), collapsed_slice_dims=(0,),
                                  start_index_map=(0,))


def _shuffle(p, idx):
    return lax.gather(p, idx[:, None], _GDN, (1,),
                      mode=lax.GatherScatterMode.PROMISE_IN_BOUNDS)


def _lanesum(p, lane):
    for sh in (8, 4, 2, 1):
        p = p + _shuffle(p, lane ^ sh)
    return p


def _mink_sign(shape):
    col = lax.broadcasted_iota(jnp.int32, shape, 1)
    return jnp.where(col == D - 1, -1.0, 1.0).astype(jnp.float32)


def _row_normalize(a):
    at = a[:, D - 1:D]
    ss = jnp.maximum(jnp.sum(a * a, axis=1, keepdims=True) - at * at, 0.0)
    inv = 1.0 / jnp.maximum(jnp.sqrt(ss), EPS)
    col = lax.broadcasted_iota(jnp.int32, a.shape, 1)
    return jnp.where(col == D - 1, a, a * inv)


def _prep_body(x_ref, wq_ref, bq_ref, wk_ref, bk_ref, wv_ref, bv_ref,
               qn_ref, knm_ref, val_ref, cr_ref):
    x = x_ref[...]
    sgn = _mink_sign((1, D))
    a, b, c, d = x[0:1], x[1:2], x[2:3], x[3:4]
    ac = jnp.sum(a * c * sgn)
    bd = jnp.sum(b * d * sgn)
    ad = jnp.sum(a * d * sgn)
    bc = jnp.sum(b * c * sgn)
    cr_ref[...] = jnp.reshape((ac * bd) / (ad * bc + EPS), (1, 1))

    xp = _row_normalize(x)
    q = jnp.dot(xp, wq_ref[...], preferred_element_type=jnp.float32) + bq_ref[...]
    k = jnp.dot(xp, wk_ref[...], preferred_element_type=jnp.float32) + bk_ref[...]
    v = jnp.dot(xp, wv_ref[...], preferred_element_type=jnp.float32) + bv_ref[...]
    qn_ref[...] = _row_normalize(q)
    kn = _row_normalize(k)
    col = lax.broadcasted_iota(jnp.int32, kn.shape, 1)
    knm_ref[...] = jnp.where(col == D - 1, -kn, kn) * SCALE
    val_ref[...] = v


@functools.partial(jax.jit, static_argnums=())
def _prep(x, Wq, bq, Wk, bk, Wv, bv):
    return pl.pallas_call(
        _prep_body,
        out_shape=[
            jax.ShapeDtypeStruct((N, D), jnp.float32),
            jax.ShapeDtypeStruct((N, D), jnp.float32),
            jax.ShapeDtypeStruct((N, D), jnp.float32),
            jax.ShapeDtypeStruct((1, 1), jnp.float32),
        ],
    )(x, Wq, bq, Wk, bk, Wv, bv)


def _edge_kernel(qn_hbm, knm_hbm, val_hbm, eidx_hbm,
                 acc_hbm, sums_hbm,
                 acc_sp,
                 ic0, ic1, ic2, ic3, ic4, ic5,
                 qb0, kb0, qb1, kb1, vb0, vb1, vb2, sbuf,
                 si0, si1, si2, si3, si4, si5,
                 sq0, sk0, sq1, sk1, sv0, sv1, sv2,
                 ss0, ss1, ss2):
    cid = lax.axis_index("c")
    sid = lax.axis_index("s")
    wid = cid * NS + sid
    ics = (ic0, ic1, ic2, ic3, ic4, ic5)
    isems = (si0, si1, si2, si3, si4, si5)
    qbufs, kbufs, vbufs = (qb0, qb1), (kb0, kb1), (vb0, vb1, vb2)
    qsems, ksems = (sq0, sq1), (sk0, sk1)
    vsems = (sv0, sv1, sv2)
    ssems = (ss0, ss1, ss2)

    zrow = jnp.zeros((16,), jnp.float32)

    def zb(i, carry):
        for j in range(D // 16):
            qb0[i, pl.ds(j * 16, 16)] = zrow
        return carry

    lax.fori_loop(0, CH, zb, 0)
    for t in range(RPT // CH):
        pltpu.sync_copy(qb0, acc_sp.at[pl.ds(sid * RPT + t * CH, CH)])
    plsc.subcore_barrier()

    lane = lax.iota(jnp.int32, 16)

    def issue_idx(g, b6):
        pltpu.async_copy(eidx_hbm.at[wid, g], ics[b6], isems[b6])

    def wait_idx(g, b6):
        pltpu.make_async_copy(eidx_hbm.at[wid, g], ics[b6], isems[b6]).wait()

    def issue_qk(g, b2, b6):
        ic = ics[b6]
        pltpu.async_copy(qn_hbm.at[ic.at[0]], qbufs[b2], qsems[b2])
        pltpu.async_copy(knm_hbm.at[ic.at[1]], kbufs[b2], ksems[b2])

    def wait_qk(g, b2, b6):
        ic = ics[b6]
        pltpu.make_async_copy(qn_hbm.at[ic.at[0]], qbufs[b2], qsems[b2]).wait()
        pltpu.make_async_copy(knm_hbm.at[ic.at[1]], kbufs[b2], ksems[b2]).wait()

    def issue_v(g, b3, b6):
        pltpu.async_copy(val_hbm.at[ics[b6].at[1]], vbufs[b3], vsems[b3])

    def wait_v(g, b3, b6):
        pltpu.make_async_copy(val_hbm.at[ics[b6].at[1]], vbufs[b3],
                              vsems[b3]).wait()

    def wait_scatter(b3, b6):
        pltpu.make_async_copy(vbufs[b3], acc_sp.at[ics[b6].at[0]],
                              ssems[b3]).wait()

    def compute(g, b2, b3, b6, valid, lsum):
        qbuf, kbuf, vbuf = qbufs[b2], kbufs[b2], vbufs[b3]

        def edot(e, ls):
            p = qbuf[e, pl.ds(0, 16)] * kbuf[e, pl.ds(0, 16)]
            for j in range(1, D // 16):
                p = p + qbuf[e, pl.ds(j * 16, 16)] * kbuf[e, pl.ds(j * 16, 16)]
            w = jnp.exp(_lanesum(p, lane))
            w = jnp.where(valid, w, 0.0)
            for j in range(D // 16):
                vbuf[e, pl.ds(j * 16, 16)] = vbuf[e, pl.ds(j * 16, 16)] * w
            return ls + w

        lsum = plsc.parallel_loop(0, CH, 1, unroll=4, carry=lsum)(edot)
        pltpu.async_copy(vbuf, acc_sp.at[ics[b6].at[0]], ssems[b3], add=True)
        return lsum

    issue_idx(0, 0)
    issue_idx(1, 1)
    issue_idx(2, 2)
    wait_idx(0, 0)
    issue_qk(0, 0, 0)
    issue_v(0, 0, 0)

    def hexa(go, lsum):
        g0 = go * 6
        for c in range(6):
            g = g0 + c
            b2, b3, b6 = c % 2, c % 3, c
            wait_idx(g + 1, (c + 1) % 6)
            issue_qk(g + 1, (c + 1) % 2, (c + 1) % 6)

            @pl.when(g >= 2)
            def _():
                wait_scatter((c + 1) % 3, (c + 4) % 6)

            issue_v(g + 1, (c + 1) % 3, (c + 1) % 6)
            wait_qk(g, b2, b6)
            wait_v(g, b3, b6)
            lsum = compute(g, b2, b3, b6, g < NCHUNK, lsum)
            issue_idx(g + 3, (c + 3) % 6)
        return lsum

    lsum = lax.fori_loop(0, NCOMP // 6, hexa, jnp.zeros((16,), jnp.float32))

    wait_idx(NCOMP + 1, (NCOMP + 1) % 6)
    wait_idx(NCOMP + 2, (NCOMP + 2) % 6)
    wait_qk(NCOMP, NCOMP % 2, NCOMP % 6)
    wait_v(NCOMP, NCOMP % 3, NCOMP % 6)
    wait_scatter((NCOMP - 2) % 3, (NCOMP - 2) % 6)
    wait_scatter((NCOMP - 1) % 3, (NCOMP - 1) % 6)

    sbuf[:] = lsum
    pltpu.sync_copy(sbuf, sums_hbm.at[pl.ds(wid * 16, 16)])

    plsc.subcore_barrier()
    for t in range(RPT // CH):
        sl = pl.ds(sid * RPT + t * CH, CH)
        pltpu.sync_copy(acc_sp.at[sl], acc_hbm.at[cid, sl])


def _edge(qn, knm, vals, rows, cols):
    mesh = plsc.VectorSubcoreMesh(core_axis_name="c", subcore_axis_name="s")
    f = functools.partial(
        pl.kernel,
        mesh=mesh,
        out_type=[
            jax.ShapeDtypeStruct((NC, NPAD, D), jnp.float32),
            jax.ShapeDtypeStruct((NW * 16,), jnp.float32),
        ],
        scratch_types=(
            [pltpu.VMEM_SHARED((NPAD, D), jnp.float32)]
            + [pltpu.VMEM((2, CH), jnp.int32)] * 6
            + [pltpu.VMEM((CH, D), jnp.float32)] * 7
            + [pltpu.VMEM((16,), jnp.float32)]
            + [pltpu.SemaphoreType.DMA] * 16
        ),
    )(_edge_kernel)
    eidx = jnp.stack([rows.reshape(NW, NCHUNK, CH),
                      cols.reshape(NW, NCHUNK, CH)], axis=2)
    eidx = jnp.pad(eidx, ((0, 0), (0, PADC - NCHUNK), (0, 0), (0, 0)))
    return f(qn, knm, vals, eidx)


def _fin_body(acc_ref, sums_ref, wo_ref, bo_ref, cr_ref, out_ref):
    A = acc_ref[0, 0:N, :] + acc_ref[1, 0:N, :]
    Z = jnp.sum(sums_ref[...][:, 0:1])
    o = (jnp.dot(A, wo_ref[...], preferred_element_type=jnp.float32) * (1.0 / Z)
         + bo_ref[...])
    sgn = _mink_sign((1, D))
    a, b, c, d = o[0:1], o[1:2], o[2:3], o[3:4]
    ac = jnp.sum(a * c * sgn)
    bd = jnp.sum(b * d * sgn)
    ad = jnp.sum(a * d * sgn)
    bc = jnp.sum(b * c * sgn)
    cr_now = (ac * bd) / (ad * bc + EPS)
    tgt = cr_ref[0, 0]
    scale = jnp.where(jnp.abs(cr_now) > EPS,
                      jnp.sqrt(jnp.abs(tgt / (cr_now + EPS))),
                      1.0)
    out_ref[...] = o * scale


def _finish(acc, sums, Wo, bo, cr):
    return pl.pallas_call(
        _fin_body,
        out_shape=jax.ShapeDtypeStruct((N, D), jnp.float32),
    )(acc, sums, Wo, bo, cr)


def kernel(x, edge_index, Wq, bq, Wk, bk, Wv, bv, Wo, bo):
    rows = edge_index[0].astype(jnp.int32)
    cols = edge_index[1].astype(jnp.int32)
    qn, knm, vals, cr = _prep(x, Wq, bq.reshape(1, D), Wk, bk.reshape(1, D),
                              Wv, bv.reshape(1, D))
    acc, sums = _edge(qn, knm, vals, rows, cols)
    return _finish(acc, sums.reshape(NW, 16), Wo, bo.reshape(1, D), cr)


# speedup vs baseline: 1.0855x; 1.0436x over previous
import functools
import math

import jax
import jax.numpy as jnp
from jax import lax
from jax.experimental import pallas as pl
from jax.experimental.pallas import tpu as pltpu
from jax.experimental.pallas import tpu_sc as plsc

EPS = 1e-9
N = 10000
D = 128
E = 320000
SCALE = 1.0 / math.sqrt(128.0)

NC = 2
NS = 16
NW = NC * NS
EPT = E // NW
CH = 40
NCHUNK = EPT // CH
NCOMP = 252
PADC = 260
NPAD = 10240
RPT = NPAD // NS


_GDN = lax.GatherDimensionNumbers(offset_dims=(), collapsed_slice_dims=(0,),
                                  start_index_map=(0,))


def _shuffle(p, idx):
    return lax.gather(p, idx[:, None], _GDN, (1,),
                      mode=lax.GatherScatterMode.PROMISE_IN_BOUNDS)


def _lanesum(p, lane):
    for sh in (8, 4, 2, 1):
        p = p + _shuffle(p, lane ^ sh)
    return p


def _mink_sign(shape):
    col = lax.broadcasted_iota(jnp.int32, shape, 1)
    return jnp.where(col == D - 1, -1.0, 1.0).astype(jnp.float32)


def _row_normalize(a):
    at = a[:, D - 1:D]
    ss = jnp.maximum(jnp.sum(a * a, axis=1, keepdims=True) - at * at, 0.0)
    inv = 1.0 / jnp.maximum(jnp.sqrt(ss), EPS)
    col = lax.broadcasted_iota(jnp.int32, a.shape, 1)
    return jnp.where(col == D - 1, a, a * inv)


def _prep_body(x_ref, wq_ref, bq_ref, wk_ref, bk_ref, wv_ref, bv_ref,
               qn_ref, knm_ref, val_ref, cr_ref):
    x = x_ref[...]
    sgn = _mink_sign((1, D))
    a, b, c, d = x[0:1], x[1:2], x[2:3], x[3:4]
    ac = jnp.sum(a * c * sgn)
    bd = jnp.sum(b * d * sgn)
    ad = jnp.sum(a * d * sgn)
    bc = jnp.sum(b * c * sgn)
    cr_ref[...] = jnp.reshape((ac * bd) / (ad * bc + EPS), (1, 1))

    xp = _row_normalize(x)
    q = jnp.dot(xp, wq_ref[...], preferred_element_type=jnp.float32) + bq_ref[...]
    k = jnp.dot(xp, wk_ref[...], preferred_element_type=jnp.float32) + bk_ref[...]
    v = jnp.dot(xp, wv_ref[...], preferred_element_type=jnp.float32) + bv_ref[...]
    qn_ref[...] = _row_normalize(q)
    kn = _row_normalize(k)
    col = lax.broadcasted_iota(jnp.int32, kn.shape, 1)
    knm_ref[...] = jnp.where(col == D - 1, -kn, kn) * SCALE
    val_ref[...] = v


@functools.partial(jax.jit, static_argnums=())
def _prep(x, Wq, bq, Wk, bk, Wv, bv):
    return pl.pallas_call(
        _prep_body,
        out_shape=[
            jax.ShapeDtypeStruct((N, D), jnp.float32),
            jax.ShapeDtypeStruct((N, D), jnp.float32),
            jax.ShapeDtypeStruct((N, D), jnp.float32),
            jax.ShapeDtypeStruct((1, 1), jnp.float32),
        ],
    )(x, Wq, bq, Wk, bk, Wv, bv)


def _edge_kernel(qn_hbm, knm_hbm, val_hbm, rows_hbm, cols_hbm,
                 acc_hbm, sums_hbm,
                 acc_sp,
                 ri0, ri1, ri2, ri3, ri4, ri5,
                 ci0, ci1, ci2, ci3, ci4, ci5,
                 qb0, kb0, qb1, kb1, vb0, vb1, vb2, sbuf,
                 si0, si1, si2, si3, si4, si5,
                 sj0, sj1, sj2, sj3, sj4, sj5,
                 sq0, sk0, sq1, sk1, sv0, sv1, sv2,
                 ss0, ss1, ss2):
    cid = lax.axis_index("c")
    sid = lax.axis_index("s")
    wid = cid * NS + sid
    rbufs = (ri0, ri1, ri2, ri3, ri4, ri5)
    cbufs = (ci0, ci1, ci2, ci3, ci4, ci5)
    rsems = (si0, si1, si2, si3, si4, si5)
    csems = (sj0, sj1, sj2, sj3, sj4, sj5)
    qbufs, kbufs, vbufs = (qb0, qb1), (kb0, kb1), (vb0, vb1, vb2)
    qsems, ksems = (sq0, sq1), (sk0, sk1)
    vsems = (sv0, sv1, sv2)
    ssems = (ss0, ss1, ss2)

    zrow = jnp.zeros((16,), jnp.float32)

    def zb(i, carry):
        for j in range(D // 16):
            qb0[i, pl.ds(j * 16, 16)] = zrow
        return carry

    lax.fori_loop(0, CH, zb, 0)
    for t in range(RPT // CH):
        pltpu.sync_copy(qb0, acc_sp.at[pl.ds(sid * RPT + t * CH, CH)])
    plsc.subcore_barrier()

    lane = lax.iota(jnp.int32, 16)

    def issue_idx(g, b6):
        pltpu.async_copy(rows_hbm.at[wid, g], rbufs[b6], rsems[b6])
        pltpu.async_copy(cols_hbm.at[wid, g], cbufs[b6], csems[b6])

    def wait_idx(g, b6):
        pltpu.make_async_copy(rows_hbm.at[wid, g], rbufs[b6], rsems[b6]).wait()
        pltpu.make_async_copy(cols_hbm.at[wid, g], cbufs[b6], csems[b6]).wait()

    def issue_qk(g, b2, b6):
        pltpu.async_copy(qn_hbm.at[rbufs[b6]], qbufs[b2], qsems[b2])
        pltpu.async_copy(knm_hbm.at[cbufs[b6]], kbufs[b2], ksems[b2])

    def wait_qk(g, b2, b6):
        pltpu.make_async_copy(qn_hbm.at[rbufs[b6]], qbufs[b2], qsems[b2]).wait()
        pltpu.make_async_copy(knm_hbm.at[cbufs[b6]], kbufs[b2], ksems[b2]).wait()

    def issue_v(g, b3, b6):
        pltpu.async_copy(val_hbm.at[cbufs[b6]], vbufs[b3], vsems[b3])

    def wait_v(g, b3, b6):
        pltpu.make_async_copy(val_hbm.at[cbufs[b6]], vbufs[b3],
                              vsems[b3]).wait()

    def wait_scatter(b3, b6):
        pltpu.make_async_copy(vbufs[b3], acc_sp.at[rbufs[b6]],
                              ssems[b3]).wait()

    def compute(g, b2, b3, b6, valid, lsum):
        qbuf, kbuf, vbuf = qbufs[b2], kbufs[b2], vbufs[b3]

        def edot(e, ls):
            p = qbuf[e, pl.ds(0, 16)] * kbuf[e, pl.ds(0, 16)]
            for j in range(1, D // 16):
                p = p + qbuf[e, pl.ds(j * 16, 16)] * kbuf[e, pl.ds(j * 16, 16)]
            w = jnp.exp(_lanesum(p, lane))
            w = jnp.where(valid, w, 0.0)
            for j in range(D // 16):
                vbuf[e, pl.ds(j * 16, 16)] = vbuf[e, pl.ds(j * 16, 16)] * w
            return ls + w

        lsum = plsc.parallel_loop(0, CH, 1, unroll=4, carry=lsum)(edot)
        pltpu.async_copy(vbuf, acc_sp.at[rbufs[b6]], ssems[b3], add=True)
        return lsum

    issue_idx(0, 0)
    issue_idx(1, 1)
    issue_idx(2, 2)
    wait_idx(0, 0)
    issue_qk(0, 0, 0)
    issue_v(0, 0, 0)

    def hexa(go, lsum):
        g0 = go * 6
        for c in range(6):
            g = g0 + c
            b2, b3, b6 = c % 2, c % 3, c
            wait_idx(g + 1, (c + 1) % 6)
            issue_qk(g + 1, (c + 1) % 2, (c + 1) % 6)

            @pl.when(g >= 2)
            def _():
                wait_scatter((c + 1) % 3, (c + 4) % 6)

            issue_v(g + 1, (c + 1) % 3, (c + 1) % 6)
            wait_qk(g, b2, b6)
            wait_v(g, b3, b6)
            lsum = compute(g, b2, b3, b6, g < NCHUNK, lsum)
            issue_idx(g + 3, (c + 3) % 6)
        return lsum

    lsum = lax.fori_loop(0, NCOMP // 6, hexa, jnp.zeros((16,), jnp.float32))

    wait_idx(NCOMP + 1, (NCOMP + 1) % 6)
    wait_idx(NCOMP + 2, (NCOMP + 2) % 6)
    wait_qk(NCOMP, NCOMP % 2, NCOMP % 6)
    wait_v(NCOMP, NCOMP % 3, NCOMP % 6)
    wait_scatter((NCOMP - 2) % 3, (NCOMP - 2) % 6)
    wait_scatter((NCOMP - 1) % 3, (NCOMP - 1) % 6)

    sbuf[:] = lsum
    pltpu.sync_copy(sbuf, sums_hbm.at[pl.ds(wid * 16, 16)])

    plsc.subcore_barrier()
    for t in range(RPT // CH):
        sl = pl.ds(sid * RPT + t * CH, CH)
        pltpu.sync_copy(acc_sp.at[sl], acc_hbm.at[cid, sl])


def _edge(qn, knm, vals, rows, cols):
    mesh = plsc.VectorSubcoreMesh(core_axis_name="c", subcore_axis_name="s")
    f = functools.partial(
        pl.kernel,
        mesh=mesh,
        out_type=[
            jax.ShapeDtypeStruct((NC, NPAD, D), jnp.float32),
            jax.ShapeDtypeStruct((NW * 16,), jnp.float32),
        ],
        scratch_types=(
            [pltpu.VMEM_SHARED((NPAD, D), jnp.float32)]
            + [pltpu.VMEM((CH,), jnp.int32)] * 12
            + [pltpu.VMEM((CH, D), jnp.float32)] * 7
            + [pltpu.VMEM((16,), jnp.float32)]
            + [pltpu.SemaphoreType.DMA] * 22
        ),
    )(_edge_kernel)
    pad = ((0, 0), (0, PADC - NCHUNK), (0, 0))
    return f(qn, knm, vals,
             jnp.pad(rows.reshape(NW, NCHUNK, CH), pad),
             jnp.pad(cols.reshape(NW, NCHUNK, CH), pad))


def _fin_body(acc_ref, sums_ref, wo_ref, bo_ref, cr_ref, out_ref):
    A = acc_ref[0, 0:N, :] + acc_ref[1, 0:N, :]
    Z = jnp.sum(sums_ref[...][:, 0:1])
    o = (jnp.dot(A, wo_ref[...], preferred_element_type=jnp.float32) * (1.0 / Z)
         + bo_ref[...])
    sgn = _mink_sign((1, D))
    a, b, c, d = o[0:1], o[1:2], o[2:3], o[3:4]
    ac = jnp.sum(a * c * sgn)
    bd = jnp.sum(b * d * sgn)
    ad = jnp.sum(a * d * sgn)
    bc = jnp.sum(b * c * sgn)
    cr_now = (ac * bd) / (ad * bc + EPS)
    tgt = cr_ref[0, 0]
    scale = jnp.where(jnp.abs(cr_now) > EPS,
                      jnp.sqrt(jnp.abs(tgt / (cr_now + EPS))),
                      1.0)
    out_ref[...] = o * scale


def _finish(acc, sums, Wo, bo, cr):
    return pl.pallas_call(
        _fin_body,
        out_shape=jax.ShapeDtypeStruct((N, D), jnp.float32),
    )(acc, sums, Wo, bo, cr)


def kernel(x, edge_index, Wq, bq, Wk, bk, Wv, bv, Wo, bo):
    rows = edge_index[0].astype(jnp.int32)
    cols = edge_index[1].astype(jnp.int32)
    qn, knm, vals, cr = _prep(x, Wq, bq.reshape(1, D), Wk, bk.reshape(1, D),
                              Wv, bv.reshape(1, D))
    acc, sums = _edge(qn, knm, vals, rows, cols)
    return _finish(acc, sums.reshape(NW, 16), Wo, bo.reshape(1, D), cr)
